# baseline (speedup 1.0000x reference)
import functools

import jax
import jax.numpy as jnp
from jax import lax
from jax.experimental import pallas as pl
from jax.experimental.pallas import tpu as pltpu
from jax.experimental.pallas import tpu_sc as plsc

_BETA = 15.0
_BATCH = 16384
_D = 16
_L = 128
_RPL = _L // _D
_N_USERS = 1000000
_N_ITEMS = 100000
_NW = 32
_BPW = _BATCH // _NW
_CHUNK = 128
_NCHUNK = _BPW // _CHUNK
_GPC = _CHUNK // 16


def _sc_kernel(user_hbm, item_hbm, ideal_hbm, mid_hbm, spread_hbm, out_hbm,
               uhi_v, ihi_v, uoff_v, ioff_v, x_v, m_v, s_v, out_v, sem):
    wid = lax.axis_index("s") * 2 + lax.axis_index("c")
    base = wid * _BPW

    pltpu.sync_copy(user_hbm.at[pl.ds(base, _BPW)], uhi_v)
    pltpu.sync_copy(item_hbm.at[pl.ds(base, _BPW)], ihi_v)

    def split(k, _):
        sl = pl.ds(k * 16, 16)
        u = uhi_v[sl]
        t = ihi_v[sl]
        uoff_v[sl] = (u & (_RPL - 1)) * _D
        ioff_v[sl] = (t & (_RPL - 1)) * _D
        uhi_v[sl] = u >> 3
        ihi_v[sl] = t >> 3
        return _

    lax.fori_loop(0, _BPW // 16, split, None)

    lane = lax.iota(jnp.int32, 16)

    def chunk(c, _):
        cbase = c * _CHUNK
        cp_x = pltpu.async_copy(
            ideal_hbm.at[uhi_v.at[pl.ds(cbase, _CHUNK)]], x_v, sem)
        cp_m = pltpu.async_copy(
            mid_hbm.at[ihi_v.at[pl.ds(cbase, _CHUNK)]], m_v, sem)
        cp_s = pltpu.async_copy(
            spread_hbm.at[ihi_v.at[pl.ds(cbase, _CHUNK)]], s_v, sem)
        cp_x.wait()
        cp_m.wait()
        cp_s.wait()

        def group(g, _):
            rows = g * 16 + lane
            uoff = plsc.load_gather(uoff_v, [cbase + rows])
            ioff = plsc.load_gather(ioff_v, [cbase + rows])
            acc = jnp.zeros((16,), jnp.float32)
            for d in range(_D):
                xv = plsc.load_gather(x_v, [rows, uoff + d])
                mv = plsc.load_gather(m_v, [rows, ioff + d])
                sv = plsc.load_gather(s_v, [rows, ioff + d])
                acc = acc + (xv - mv) * sv
            out_v[pl.ds(cbase + g * 16, 16)] = acc * _BETA
            return _

        lax.fori_loop(0, _GPC, group, None)
        return _

    lax.fori_loop(0, _NCHUNK, chunk, None)

    pltpu.sync_copy(out_v, out_hbm.at[pl.ds(base, _BPW)])


@jax.jit
def kernel(user_idx, item_idx, ideal_points, vote_midpoints, vote_spreads):
    mesh = plsc.VectorSubcoreMesh(core_axis_name="c", subcore_axis_name="s")
    run = functools.partial(
        pl.kernel,
        mesh=mesh,
        out_type=jax.ShapeDtypeStruct((_BATCH,), jnp.float32),
        scratch_types=[
            pltpu.VMEM((_BPW,), jnp.int32),
            pltpu.VMEM((_BPW,), jnp.int32),
            pltpu.VMEM((_BPW,), jnp.int32),
            pltpu.VMEM((_BPW,), jnp.int32),
            pltpu.VMEM((_CHUNK, _L), jnp.float32),
            pltpu.VMEM((_CHUNK, _L), jnp.float32),
            pltpu.VMEM((_CHUNK, _L), jnp.float32),
            pltpu.VMEM((_BPW,), jnp.float32),
            pltpu.SemaphoreType.DMA,
        ],
        compiler_params=pltpu.CompilerParams(
            needs_layout_passes=False, use_tc_tiling_on_sc=False),
    )(_sc_kernel)
    return run(user_idx.astype(jnp.int32), item_idx.astype(jnp.int32),
               ideal_points.reshape(_N_USERS // _RPL, _L),
               vote_midpoints.reshape(_N_ITEMS // _RPL, _L),
               vote_spreads.reshape(_N_ITEMS // _RPL, _L))
